# baseline (speedup 1.0000x reference)
import functools

import jax
import jax.numpy as jnp
from jax import lax
from jax.experimental import pallas as pl
from jax.experimental.pallas import tpu as pltpu
from jax.experimental.pallas import tpu_sc as plsc

K = 64
THRESHOLD = 0.5
T0_BITS = 0x3F780000
HALF_BITS = 0x3F000000
ONE_BITS = 0x3F800000
CAP = 256

NW = 32


def _row_pass(buf, r, fn, init, nvreg):
    def body(j, carry):
        for u in range(8):
            v = buf[r, pl.ds((j * 8 + u) * 16, 16)]
            carry = fn(v, carry)
        return carry

    return lax.fori_loop(0, nvreg // 8, body, init)


def _masked_sum_scalar(buf, r, nvreg):
    def fn(v, acc):
        return acc + jnp.where(v >= THRESHOLD, v, 0.0)

    acc = _row_pass(buf, r, fn, jnp.zeros((16,), jnp.float32), nvreg)
    return jnp.sum(acc)


def _fallback_topk_sum(buf, r, nvreg):
    def fn(v, carry):
        s, c = carry
        m = v >= THRESHOLD
        return s + jnp.where(m, v, 0.0), c + m.astype(jnp.int32)

    s5, c5 = _row_pass(buf, r, fn, (jnp.zeros((16,), jnp.float32),
                                    jnp.zeros((16,), jnp.int32)), nvreg)
    count5 = jnp.sum(c5)
    sum5 = jnp.sum(s5)

    def search(_):
        lo, hi = HALF_BITS, ONE_BITS - 1

        def bit_iter(_, carry):
            lo, hi = carry
            mid = lo + ((hi - lo + 1) >> 1)

            def fn(v, cnt):
                b = lax.bitcast_convert_type(v, jnp.int32)
                return cnt + jnp.where(b >= mid, 1, 0)

            cnt = jnp.sum(_row_pass(buf, r, fn, jnp.zeros((16,), jnp.int32),
                                    nvreg))
            ge = cnt >= K
            return jnp.where(ge, mid, lo), jnp.where(ge, hi, mid - 1)

        lo, hi = lax.fori_loop(0, 23, bit_iter, (lo, hi))
        kth = lax.bitcast_convert_type(lo, jnp.float32)

        def fn2(v, carry):
            s, c = carry
            b = lax.bitcast_convert_type(v, jnp.int32)
            g = b > lo
            return s + jnp.where(g, v, 0.0), c + g.astype(jnp.int32)

        s, c = _row_pass(buf, r, fn2, (jnp.zeros((16,), jnp.float32),
                                       jnp.zeros((16,), jnp.int32)), nvreg)
        return jnp.sum(s) + (K - jnp.sum(c)).astype(jnp.float32) * kth

    return jnp.where(count5 < K, sum5, search(None))


def _sc_deg_body(feat_hbm, deg_hbm, rowbuf, cand, degbuf, *, nvreg, rpw):
    wid = lax.axis_index("s") * 2 + lax.axis_index("c")
    nor0 = wid * rpw
    abn0 = NW * rpw + wid * rpw
    lanes = lax.iota(jnp.int32, 16)
    t0f = lax.bitcast_convert_type(jnp.full((16,), T0_BITS, jnp.int32),
                                   jnp.float32)

    nbatch = rpw // 16

    def nor_batch(batch, _):
        pltpu.sync_copy(feat_hbm.at[pl.ds(nor0 + batch * 16, 16)], rowbuf)

        def nor_row(r, degs):
            s = _masked_sum_scalar(rowbuf, r, nvreg)
            return jnp.where(lanes == r, s, degs)

        degs = lax.fori_loop(0, 16, nor_row, jnp.zeros((16,), jnp.float32))
        degbuf[pl.ds(batch * 16, 16)] = degs
        return 0

    lax.fori_loop(0, nbatch, nor_batch, 0)

    def abn_batch(batch, _):
        pltpu.sync_copy(feat_hbm.at[pl.ds(abn0 + batch * 16, 16)], rowbuf)

        def zero_it(j, _):
            for u in range(8):
                cand[j * 8 + u] = jnp.zeros((16,), jnp.float32)
            return 0

        lax.fori_loop(0, CAP // 8, zero_it, 0)

        def abn_row(r, carry):
            def fn(v, off):
                m = v >= t0f
                pc = plsc.all_reduce_population_count(m)
                pref = plsc.cumsum(m.astype(jnp.int32))
                pos = off + pref - 1
                ok = m & (pos < CAP)
                posc = jnp.clip(pos, 0, CAP - 1)
                plsc.store_scatter(cand, [posc, lanes], v, mask=ok)
                return off + pc

            off = _row_pass(rowbuf, r, fn, jnp.zeros((16,), jnp.int32), nvreg)
            m_r = off[0]
            m_vec, fb_vec = carry
            bad_r = (m_r < K) | (m_r > CAP)
            fb = lax.cond(bad_r,
                          lambda: _fallback_topk_sum(rowbuf, r, nvreg),
                          lambda: jnp.float32(0.0))
            m_vec = jnp.where(lanes == r, m_r, m_vec)
            fb_vec = jnp.where(lanes == r, fb, fb_vec)
            return m_vec, fb_vec

        m_vec, fb_vec = lax.fori_loop(
            0, 16, abn_row,
            (jnp.zeros((16,), jnp.int32), jnp.zeros((16,), jnp.float32)))

        groups = (jnp.minimum(jnp.max(m_vec), CAP) + 7) >> 3

        def cand_pass(fn, init):
            def body(j, carry):
                for u in range(8):
                    row = cand[j * 8 + u]
                    carry = fn(row, carry)
                return carry

            return lax.fori_loop(0, groups, body, init)

        def bit_iter(_, carry):
            lo, hi = carry
            mid = lo + ((hi - lo + 1) >> 1)

            def cnt_fn(row, cnt):
                b = lax.bitcast_convert_type(row, jnp.int32)
                return cnt + jnp.where(b >= mid, 1, 0)

            cnt = cand_pass(cnt_fn, jnp.zeros((16,), jnp.int32))
            ge = cnt >= K
            return jnp.where(ge, mid, lo), jnp.where(ge, hi, mid - 1)

        lo, hi = lax.fori_loop(0, 19, bit_iter,
                               (jnp.full((16,), T0_BITS, jnp.int32),
                                jnp.full((16,), ONE_BITS - 1, jnp.int32)))
        kth = lax.bitcast_convert_type(lo, jnp.float32)

        def corr_fn(row, carry):
            s, c = carry
            b = lax.bitcast_convert_type(row, jnp.int32)
            g = b > lo
            return s + jnp.where(g, row, 0.0), c + g.astype(jnp.int32)

        s, c = cand_pass(corr_fn, (jnp.zeros((16,), jnp.float32),
                                   jnp.zeros((16,), jnp.int32)))
        deg_b = s + (K - c).astype(jnp.float32) * kth

        bad = (m_vec < K) | (m_vec > CAP)
        degbuf[pl.ds(rpw + batch * 16, 16)] = jnp.where(bad, fb_vec, deg_b)
        return 0

    lax.fori_loop(0, nbatch, abn_batch, 0)

    pltpu.sync_copy(degbuf.at[pl.ds(0, rpw)],
                    deg_hbm.at[pl.ds(nor0, rpw)])
    pltpu.sync_copy(degbuf.at[pl.ds(rpw, rpw)],
                    deg_hbm.at[pl.ds(abn0, rpw)])


def _loss_kernel(deg_ref, out_ref, *, b2):
    deg = deg_ref[...]
    b, c = deg.shape
    mean = jnp.mean(deg, axis=1, keepdims=True)
    d = deg - mean
    var = jnp.sum(d * d, axis=1, keepdims=True) / (c - 1)
    sign = jnp.where(lax.broadcasted_iota(jnp.int32, (b, 1), 0) < b2, 1.0, -1.0)
    out_ref[...] = jnp.sum(var * sign, axis=(0, 1), keepdims=True) / b2


def kernel(features):
    b, c, t = features.shape
    b2 = b // 2
    nrows = b * c
    feat2d = jnp.reshape(features, (nrows, t))

    rpw = (nrows // 2) // NW
    assert rpw % 16 == 0 and t % 128 == 0
    mesh = plsc.VectorSubcoreMesh(core_axis_name="c", subcore_axis_name="s",
                                  num_cores=2, num_subcores=16)
    sc_deg = pl.kernel(
        functools.partial(_sc_deg_body, nvreg=t // 16, rpw=rpw),
        out_type=jax.ShapeDtypeStruct((nrows,), jnp.float32),
        mesh=mesh,
        scratch_types=[
            pltpu.VMEM((16, t), jnp.float32),
            pltpu.VMEM((CAP, 16), jnp.float32),
            pltpu.VMEM((2 * rpw,), jnp.float32),
        ],
        compiler_params=pltpu.CompilerParams(needs_layout_passes=False),
    )
    deg = jnp.reshape(sc_deg(feat2d), (b, c))

    loss = pl.pallas_call(
        functools.partial(_loss_kernel, b2=b2),
        out_shape=jax.ShapeDtypeStruct((1, 1), jnp.float32),
    )(deg)
    return jnp.reshape(loss, ())


# speedup vs baseline: 1.0273x; 1.0273x over previous
import functools

import jax
import jax.numpy as jnp
from jax import lax
from jax.experimental import pallas as pl
from jax.experimental.pallas import tpu as pltpu
from jax.experimental.pallas import tpu_sc as plsc

K = 64
THRESHOLD = 0.5
T0_BITS = 0x3F780000
HALF_BITS = 0x3F000000
ONE_BITS = 0x3F800000
CAP = 256

NW = 32


def _row_pass(buf, r, fn, init, nvreg):
    def body(j, carry):
        for u in range(8):
            v = buf[r, pl.ds((j * 8 + u) * 16, 16)]
            carry = fn(v, carry)
        return carry

    return lax.fori_loop(0, nvreg // 8, body, init)


def _row_pass8(buf, r, fn, init1, nvreg):
    def body(j, accs):
        return tuple(
            fn(buf[r, pl.ds((j * 8 + u) * 16, 16)], acc)
            for u, acc in enumerate(accs)
        )

    accs = lax.fori_loop(0, nvreg // 8, body, (init1,) * 8)
    out = accs[0]
    for a in accs[1:]:
        out = jax.tree.map(lambda x, y: x + y, out, a)
    return out


def _masked_sum_scalar(buf, r, nvreg):
    def fn(v, acc):
        return acc + jnp.where(v >= THRESHOLD, v, 0.0)

    acc = _row_pass8(buf, r, fn, jnp.zeros((16,), jnp.float32), nvreg)
    return jnp.sum(acc)


def _fallback_topk_sum(buf, r, nvreg):
    def fn(v, carry):
        s, c = carry
        m = v >= THRESHOLD
        return s + jnp.where(m, v, 0.0), c + m.astype(jnp.int32)

    s5, c5 = _row_pass(buf, r, fn, (jnp.zeros((16,), jnp.float32),
                                    jnp.zeros((16,), jnp.int32)), nvreg)
    count5 = jnp.sum(c5)
    sum5 = jnp.sum(s5)

    def search(_):
        lo, hi = HALF_BITS, ONE_BITS - 1

        def bit_iter(_, carry):
            lo, hi = carry
            mid = lo + ((hi - lo + 1) >> 1)

            def fn(v, cnt):
                b = lax.bitcast_convert_type(v, jnp.int32)
                return cnt + jnp.where(b >= mid, 1, 0)

            cnt = jnp.sum(_row_pass(buf, r, fn, jnp.zeros((16,), jnp.int32),
                                    nvreg))
            ge = cnt >= K
            return jnp.where(ge, mid, lo), jnp.where(ge, hi, mid - 1)

        lo, hi = lax.fori_loop(0, 23, bit_iter, (lo, hi))
        kth = lax.bitcast_convert_type(lo, jnp.float32)

        def fn2(v, carry):
            s, c = carry
            b = lax.bitcast_convert_type(v, jnp.int32)
            g = b > lo
            return s + jnp.where(g, v, 0.0), c + g.astype(jnp.int32)

        s, c = _row_pass(buf, r, fn2, (jnp.zeros((16,), jnp.float32),
                                       jnp.zeros((16,), jnp.int32)), nvreg)
        return jnp.sum(s) + (K - jnp.sum(c)).astype(jnp.float32) * kth

    return jnp.where(count5 < K, sum5, search(None))


def _sc_deg_body(feat_hbm, deg_hbm, rowbuf, cand, degbuf, *, nvreg, rpw):
    wid = lax.axis_index("s") * 2 + lax.axis_index("c")
    nor0 = wid * rpw
    abn0 = NW * rpw + wid * rpw
    lanes = lax.iota(jnp.int32, 16)
    t0f = lax.bitcast_convert_type(jnp.full((16,), T0_BITS, jnp.int32),
                                   jnp.float32)

    nbatch = rpw // 16

    def nor_batch(batch, _):
        pltpu.sync_copy(feat_hbm.at[pl.ds(nor0 + batch * 16, 16)], rowbuf)

        def nor_row(r, degs):
            s = _masked_sum_scalar(rowbuf, r, nvreg)
            return jnp.where(lanes == r, s, degs)

        degs = lax.fori_loop(0, 16, nor_row, jnp.zeros((16,), jnp.float32))
        degbuf[pl.ds(batch * 16, 16)] = degs
        return 0

    lax.fori_loop(0, nbatch, nor_batch, 0)

    def abn_batch(batch, _):
        pltpu.sync_copy(feat_hbm.at[pl.ds(abn0 + batch * 16, 16)], rowbuf)

        def zero_it(j, _):
            for u in range(8):
                cand[j * 8 + u] = jnp.zeros((16,), jnp.float32)
            return 0

        lax.fori_loop(0, CAP // 8, zero_it, 0)

        def abn_row(r, carry):
            def fn(v, off):
                m = v >= t0f
                pc = plsc.all_reduce_population_count(m)
                pref = plsc.cumsum(m.astype(jnp.int32))
                pos = off + pref - 1
                ok = m & (pos < CAP)
                posc = jnp.clip(pos, 0, CAP - 1)
                plsc.store_scatter(cand, [posc, lanes], v, mask=ok)
                return off + pc

            off = _row_pass(rowbuf, r, fn, jnp.zeros((16,), jnp.int32), nvreg)
            m_r = off[0]
            m_vec, fb_vec = carry
            bad_r = (m_r < K) | (m_r > CAP)
            fb = lax.cond(bad_r,
                          lambda: _fallback_topk_sum(rowbuf, r, nvreg),
                          lambda: jnp.float32(0.0))
            m_vec = jnp.where(lanes == r, m_r, m_vec)
            fb_vec = jnp.where(lanes == r, fb, fb_vec)
            return m_vec, fb_vec

        m_vec, fb_vec = lax.fori_loop(
            0, 16, abn_row,
            (jnp.zeros((16,), jnp.int32), jnp.zeros((16,), jnp.float32)))

        groups = (jnp.minimum(jnp.max(m_vec), CAP) + 7) >> 3

        def cand_pass(fn, init):
            def body(j, accs):
                return tuple(
                    fn(cand[j * 8 + u], acc) for u, acc in enumerate(accs)
                )

            accs = lax.fori_loop(0, groups, body, (init,) * 8)
            out = accs[0]
            for a in accs[1:]:
                out = jax.tree.map(lambda x, y: x + y, out, a)
            return out

        def bit_iter(_, carry):
            lo, hi = carry
            mid = lo + ((hi - lo + 1) >> 1)

            def cnt_fn(row, cnt):
                b = lax.bitcast_convert_type(row, jnp.int32)
                return cnt + jnp.where(b >= mid, 1, 0)

            cnt = cand_pass(cnt_fn, jnp.zeros((16,), jnp.int32))
            ge = cnt >= K
            return jnp.where(ge, mid, lo), jnp.where(ge, hi, mid - 1)

        lo, hi = lax.fori_loop(0, 19, bit_iter,
                               (jnp.full((16,), T0_BITS, jnp.int32),
                                jnp.full((16,), ONE_BITS - 1, jnp.int32)))
        kth = lax.bitcast_convert_type(lo, jnp.float32)

        def corr_fn(row, carry):
            s, c = carry
            b = lax.bitcast_convert_type(row, jnp.int32)
            g = b > lo
            return s + jnp.where(g, row, 0.0), c + g.astype(jnp.int32)

        s, c = cand_pass(corr_fn, (jnp.zeros((16,), jnp.float32),
                                   jnp.zeros((16,), jnp.int32)))
        deg_b = s + (K - c).astype(jnp.float32) * kth

        bad = (m_vec < K) | (m_vec > CAP)
        degbuf[pl.ds(rpw + batch * 16, 16)] = jnp.where(bad, fb_vec, deg_b)
        return 0

    lax.fori_loop(0, nbatch, abn_batch, 0)

    pltpu.sync_copy(degbuf.at[pl.ds(0, rpw)],
                    deg_hbm.at[pl.ds(nor0, rpw)])
    pltpu.sync_copy(degbuf.at[pl.ds(rpw, rpw)],
                    deg_hbm.at[pl.ds(abn0, rpw)])


def _loss_kernel(deg_ref, out_ref, *, b2):
    deg = deg_ref[...]
    b, c = deg.shape
    mean = jnp.mean(deg, axis=1, keepdims=True)
    d = deg - mean
    var = jnp.sum(d * d, axis=1, keepdims=True) / (c - 1)
    sign = jnp.where(lax.broadcasted_iota(jnp.int32, (b, 1), 0) < b2, 1.0, -1.0)
    out_ref[...] = jnp.sum(var * sign, axis=(0, 1), keepdims=True) / b2


def kernel(features):
    b, c, t = features.shape
    b2 = b // 2
    nrows = b * c
    feat2d = jnp.reshape(features, (nrows, t))

    rpw = (nrows // 2) // NW
    assert rpw % 16 == 0 and t % 128 == 0
    mesh = plsc.VectorSubcoreMesh(core_axis_name="c", subcore_axis_name="s",
                                  num_cores=2, num_subcores=16)
    sc_deg = pl.kernel(
        functools.partial(_sc_deg_body, nvreg=t // 16, rpw=rpw),
        out_type=jax.ShapeDtypeStruct((nrows,), jnp.float32),
        mesh=mesh,
        scratch_types=[
            pltpu.VMEM((16, t), jnp.float32),
            pltpu.VMEM((CAP, 16), jnp.float32),
            pltpu.VMEM((2 * rpw,), jnp.float32),
        ],
        compiler_params=pltpu.CompilerParams(needs_layout_passes=False),
    )
    deg = jnp.reshape(sc_deg(feat2d), (b, c))

    loss = pl.pallas_call(
        functools.partial(_loss_kernel, b2=b2),
        out_shape=jax.ShapeDtypeStruct((1, 1), jnp.float32),
    )(deg)
    return jnp.reshape(loss, ())


# speedup vs baseline: 3.0896x; 3.0075x over previous
import functools

import jax
import jax.numpy as jnp
from jax import lax
from jax.experimental import pallas as pl
from jax.experimental.pallas import tpu as pltpu
from jax.experimental.pallas import tpu_sc as plsc

K = 64
THRESHOLD = 0.5
T0_BITS = 0x3F780000
HALF_BITS = 0x3F000000
ONE_BITS = 0x3F800000
CAP = 256

NW = 32


def _row_pass(buf, r, fn, init, nvreg):
    def body(j, carry):
        for u in range(8):
            v = buf[r, pl.ds((j * 8 + u) * 16, 16)]
            carry = fn(v, carry)
        return carry

    return lax.fori_loop(0, nvreg // 8, body, init)


def _row_pass8(buf, r, fn, init1, nvreg):
    def body(j, accs):
        return tuple(
            fn(buf[r, pl.ds((j * 8 + u) * 16, 16)], acc)
            for u, acc in enumerate(accs)
        )

    accs = plsc.parallel_loop(0, nvreg // 8, unroll=2,
                              carry=(init1,) * 8)(body)
    out = accs[0]
    for a in accs[1:]:
        out = jax.tree.map(lambda x, y: x + y, out, a)
    return out


def _masked_sum_scalar(buf, r, nvreg):
    def fn(v, acc):
        return acc + jnp.where(v >= THRESHOLD, v, 0.0)

    acc = _row_pass8(buf, r, fn, jnp.zeros((16,), jnp.float32), nvreg)
    return jnp.sum(acc)


def _fallback_topk_sum(buf, r, nvreg):
    def fn(v, carry):
        s, c = carry
        m = v >= THRESHOLD
        return s + jnp.where(m, v, 0.0), c + m.astype(jnp.int32)

    s5, c5 = _row_pass(buf, r, fn, (jnp.zeros((16,), jnp.float32),
                                    jnp.zeros((16,), jnp.int32)), nvreg)
    count5 = jnp.sum(c5)
    sum5 = jnp.sum(s5)

    def search(_):
        lo, hi = HALF_BITS, ONE_BITS - 1

        def bit_iter(_, carry):
            lo, hi = carry
            mid = lo + ((hi - lo + 1) >> 1)

            def fn(v, cnt):
                b = lax.bitcast_convert_type(v, jnp.int32)
                return cnt + jnp.where(b >= mid, 1, 0)

            cnt = jnp.sum(_row_pass(buf, r, fn, jnp.zeros((16,), jnp.int32),
                                    nvreg))
            ge = cnt >= K
            return jnp.where(ge, mid, lo), jnp.where(ge, hi, mid - 1)

        lo, hi = lax.fori_loop(0, 23, bit_iter, (lo, hi))
        kth = lax.bitcast_convert_type(lo, jnp.float32)

        def fn2(v, carry):
            s, c = carry
            b = lax.bitcast_convert_type(v, jnp.int32)
            g = b > lo
            return s + jnp.where(g, v, 0.0), c + g.astype(jnp.int32)

        s, c = _row_pass(buf, r, fn2, (jnp.zeros((16,), jnp.float32),
                                       jnp.zeros((16,), jnp.int32)), nvreg)
        return jnp.sum(s) + (K - jnp.sum(c)).astype(jnp.float32) * kth

    return jnp.where(count5 < K, sum5, search(None))


def _sc_deg_body(feat_hbm, deg_hbm, rowbuf, cand, degbuf, *, nvreg, rpw):
    wid = lax.axis_index("s") * 2 + lax.axis_index("c")
    nor0 = wid * rpw
    abn0 = NW * rpw + wid * rpw
    lanes = lax.iota(jnp.int32, 16)
    t0f = lax.bitcast_convert_type(jnp.full((16,), T0_BITS, jnp.int32),
                                   jnp.float32)

    nbatch = rpw // 16

    def nor_batch(batch, _):
        pltpu.sync_copy(feat_hbm.at[pl.ds(nor0 + batch * 16, 16)], rowbuf)

        def nor_row(r, degs):
            s = _masked_sum_scalar(rowbuf, r, nvreg)
            return jnp.where(lanes == r, s, degs)

        degs = lax.fori_loop(0, 16, nor_row, jnp.zeros((16,), jnp.float32))
        degbuf[pl.ds(batch * 16, 16)] = degs
        return 0

    lax.fori_loop(0, nbatch, nor_batch, 0)

    def abn_batch(batch, _):
        pltpu.sync_copy(feat_hbm.at[pl.ds(abn0 + batch * 16, 16)], rowbuf)

        @plsc.parallel_loop(0, CAP, unroll=8)
        def _zero(j):
            cand[j] = jnp.zeros((16,), jnp.float32)

        def abn_row(r, carry):
            def fn(v, off):
                m = v >= t0f
                pc = plsc.all_reduce_population_count(m)
                pref = plsc.cumsum(m.astype(jnp.int32))
                pos = off + pref - 1
                ok = m & (pos < CAP)
                posc = jnp.clip(pos, 0, CAP - 1)
                plsc.store_scatter(cand, [posc, lanes], v, mask=ok)
                return off + pc

            def fnA(j, off):
                return fn(rowbuf[r, pl.ds(j * 16, 16)], off)

            off = plsc.parallel_loop(0, nvreg, unroll=8,
                                     carry=jnp.zeros((16,), jnp.int32))(fnA)
            m_r = off[0]
            m_vec, fb_vec = carry
            bad_r = (m_r < K) | (m_r > CAP)
            fb = lax.cond(bad_r,
                          lambda: _fallback_topk_sum(rowbuf, r, nvreg),
                          lambda: jnp.float32(0.0))
            m_vec = jnp.where(lanes == r, m_r, m_vec)
            fb_vec = jnp.where(lanes == r, fb, fb_vec)
            return m_vec, fb_vec

        m_vec, fb_vec = lax.fori_loop(
            0, 16, abn_row,
            (jnp.zeros((16,), jnp.int32), jnp.zeros((16,), jnp.float32)))

        groups = (jnp.minimum(jnp.max(m_vec), CAP) + 7) >> 3

        def cand_pass(fn, init):
            def body(j, accs):
                return tuple(
                    fn(cand[j * 8 + u], acc) for u, acc in enumerate(accs)
                )

            accs = plsc.parallel_loop(0, groups, unroll=1,
                                      carry=(init,) * 8)(body)
            out = accs[0]
            for a in accs[1:]:
                out = jax.tree.map(lambda x, y: x + y, out, a)
            return out

        def bit_iter(_, carry):
            lo, hi = carry
            mid = lo + ((hi - lo + 1) >> 1)

            def cnt_fn(row, cnt):
                b = lax.bitcast_convert_type(row, jnp.int32)
                return cnt + jnp.where(b >= mid, 1, 0)

            cnt = cand_pass(cnt_fn, jnp.zeros((16,), jnp.int32))
            ge = cnt >= K
            return jnp.where(ge, mid, lo), jnp.where(ge, hi, mid - 1)

        lo, hi = lax.fori_loop(0, 19, bit_iter,
                               (jnp.full((16,), T0_BITS, jnp.int32),
                                jnp.full((16,), ONE_BITS - 1, jnp.int32)))
        kth = lax.bitcast_convert_type(lo, jnp.float32)

        def corr_fn(row, carry):
            s, c = carry
            b = lax.bitcast_convert_type(row, jnp.int32)
            g = b > lo
            return s + jnp.where(g, row, 0.0), c + g.astype(jnp.int32)

        s, c = cand_pass(corr_fn, (jnp.zeros((16,), jnp.float32),
                                   jnp.zeros((16,), jnp.int32)))
        deg_b = s + (K - c).astype(jnp.float32) * kth

        bad = (m_vec < K) | (m_vec > CAP)
        degbuf[pl.ds(rpw + batch * 16, 16)] = jnp.where(bad, fb_vec, deg_b)
        return 0

    lax.fori_loop(0, nbatch, abn_batch, 0)

    pltpu.sync_copy(degbuf.at[pl.ds(0, rpw)],
                    deg_hbm.at[pl.ds(nor0, rpw)])
    pltpu.sync_copy(degbuf.at[pl.ds(rpw, rpw)],
                    deg_hbm.at[pl.ds(abn0, rpw)])


def _loss_kernel(deg_ref, out_ref, *, b2):
    deg = deg_ref[...]
    b, c = deg.shape
    mean = jnp.mean(deg, axis=1, keepdims=True)
    d = deg - mean
    var = jnp.sum(d * d, axis=1, keepdims=True) / (c - 1)
    sign = jnp.where(lax.broadcasted_iota(jnp.int32, (b, 1), 0) < b2, 1.0, -1.0)
    out_ref[...] = jnp.sum(var * sign, axis=(0, 1), keepdims=True) / b2


def kernel(features):
    b, c, t = features.shape
    b2 = b // 2
    nrows = b * c
    feat2d = jnp.reshape(features, (nrows, t))

    rpw = (nrows // 2) // NW
    assert rpw % 16 == 0 and t % 128 == 0
    mesh = plsc.VectorSubcoreMesh(core_axis_name="c", subcore_axis_name="s",
                                  num_cores=2, num_subcores=16)
    sc_deg = pl.kernel(
        functools.partial(_sc_deg_body, nvreg=t // 16, rpw=rpw),
        out_type=jax.ShapeDtypeStruct((nrows,), jnp.float32),
        mesh=mesh,
        scratch_types=[
            pltpu.VMEM((16, t), jnp.float32),
            pltpu.VMEM((CAP, 16), jnp.float32),
            pltpu.VMEM((2 * rpw,), jnp.float32),
        ],
        compiler_params=pltpu.CompilerParams(needs_layout_passes=False),
    )
    deg = jnp.reshape(sc_deg(feat2d), (b, c))

    loss = pl.pallas_call(
        functools.partial(_loss_kernel, b2=b2),
        out_shape=jax.ShapeDtypeStruct((1, 1), jnp.float32),
    )(deg)
    return jnp.reshape(loss, ())


# speedup vs baseline: 3.8674x; 1.2518x over previous
import functools

import jax
import jax.numpy as jnp
from jax import lax
from jax.experimental import pallas as pl
from jax.experimental.pallas import tpu as pltpu
from jax.experimental.pallas import tpu_sc as plsc

K = 64
THRESHOLD = 0.5
T0_BITS = 0x3F780000
HALF_BITS = 0x3F000000
ONE_BITS = 0x3F800000
CAP = 256

NW = 32


def _row_pass(buf, r, fn, init, nvreg):
    def body(j, carry):
        for u in range(8):
            v = buf[r, pl.ds((j * 8 + u) * 16, 16)]
            carry = fn(v, carry)
        return carry

    return lax.fori_loop(0, nvreg // 8, body, init)


def _row_pass8(buf, r, fn, init1, nvreg):
    def body(j, accs):
        return tuple(
            fn(buf[r, pl.ds((j * 8 + u) * 16, 16)], acc)
            for u, acc in enumerate(accs)
        )

    accs = plsc.parallel_loop(0, nvreg // 8, unroll=2,
                              carry=(init1,) * 8)(body)
    out = accs[0]
    for a in accs[1:]:
        out = jax.tree.map(lambda x, y: x + y, out, a)
    return out


def _masked_sum_scalar(buf, r, nvreg):
    def fn(v, acc):
        return acc + jnp.where(v >= THRESHOLD, v, 0.0)

    acc = _row_pass8(buf, r, fn, jnp.zeros((16,), jnp.float32), nvreg)
    return jnp.sum(acc)


def _fallback_topk_sum(buf, r, nvreg):
    def fn(v, carry):
        s, c = carry
        m = v >= THRESHOLD
        return s + jnp.where(m, v, 0.0), c + m.astype(jnp.int32)

    s5, c5 = _row_pass(buf, r, fn, (jnp.zeros((16,), jnp.float32),
                                    jnp.zeros((16,), jnp.int32)), nvreg)
    count5 = jnp.sum(c5)
    sum5 = jnp.sum(s5)

    def search(_):
        lo, hi = HALF_BITS, ONE_BITS - 1

        def bit_iter(_, carry):
            lo, hi = carry
            mid = lo + ((hi - lo + 1) >> 1)

            def fn(v, cnt):
                b = lax.bitcast_convert_type(v, jnp.int32)
                return cnt + jnp.where(b >= mid, 1, 0)

            cnt = jnp.sum(_row_pass(buf, r, fn, jnp.zeros((16,), jnp.int32),
                                    nvreg))
            ge = cnt >= K
            return jnp.where(ge, mid, lo), jnp.where(ge, hi, mid - 1)

        lo, hi = lax.fori_loop(0, 23, bit_iter, (lo, hi))
        kth = lax.bitcast_convert_type(lo, jnp.float32)

        def fn2(v, carry):
            s, c = carry
            b = lax.bitcast_convert_type(v, jnp.int32)
            g = b > lo
            return s + jnp.where(g, v, 0.0), c + g.astype(jnp.int32)

        s, c = _row_pass(buf, r, fn2, (jnp.zeros((16,), jnp.float32),
                                       jnp.zeros((16,), jnp.int32)), nvreg)
        return jnp.sum(s) + (K - jnp.sum(c)).astype(jnp.float32) * kth

    return jnp.where(count5 < K, sum5, search(None))


def _sc_deg_body(feat_hbm, deg_hbm, rowbuf, cand, degbuf, sem0, sem1,
                 *, nvreg, rpw):
    wid = lax.axis_index("s") * 2 + lax.axis_index("c")
    nor0 = wid * rpw
    abn0 = NW * rpw + wid * rpw
    lanes = lax.iota(jnp.int32, 16)
    t0f = lax.bitcast_convert_type(jnp.full((16,), T0_BITS, jnp.int32),
                                   jnp.float32)

    nbatch = rpw // 16
    sems = (sem0, sem1)

    def cp(row0, bufi):
        return pltpu.make_async_copy(feat_hbm.at[pl.ds(row0, 8)],
                                     rowbuf.at[bufi], sems[bufi])

    cp(nor0, 0).start()
    cp(nor0 + 8, 1).start()

    def nor_batch(batch, _):
        c0 = nor0 + batch * 16
        degs = jnp.zeros((16,), jnp.float32)
        for bufi in range(2):
            cp(c0 + 8 * bufi, bufi).wait()

            def nor_row(r, degs, bufi=bufi):
                s = _masked_sum_scalar(rowbuf.at[bufi], r, nvreg)
                return jnp.where(lanes == 8 * bufi + r, s, degs)

            degs = lax.fori_loop(0, 8, nor_row, degs)

            @pl.when(batch + 1 < nbatch)
            def _prefetch(bufi=bufi):
                cp(c0 + 16 + 8 * bufi, bufi).start()

        degbuf[pl.ds(batch * 16, 16)] = degs
        return 0

    lax.fori_loop(0, nbatch, nor_batch, 0)

    cp(abn0, 0).start()
    cp(abn0 + 8, 1).start()

    def abn_batch(batch, _):
        c0 = abn0 + batch * 16

        @plsc.parallel_loop(0, CAP, unroll=8)
        def _zero(j):
            cand[j] = jnp.zeros((16,), jnp.float32)

        carry = (jnp.zeros((16,), jnp.int32), jnp.zeros((16,), jnp.float32))
        for bufi in range(2):
            cp(c0 + 8 * bufi, bufi).wait()

            def abn_row(r, carry, bufi=bufi):
                buf = rowbuf.at[bufi]
                col = lanes * 0 + (8 * bufi + r)

                def fn(v, off):
                    m = v >= t0f
                    pc = plsc.all_reduce_population_count(m)
                    pref = plsc.cumsum(m.astype(jnp.int32))
                    pos = off + pref - 1
                    ok = m & (pos < CAP)
                    posc = jnp.clip(pos, 0, CAP - 1)
                    plsc.store_scatter(cand, [posc, col], v, mask=ok)
                    return off + pc

                def fnA(j, off):
                    return fn(buf[r, pl.ds(j * 16, 16)], off)

                off = plsc.parallel_loop(0, nvreg, unroll=8,
                                         carry=jnp.zeros((16,), jnp.int32))(fnA)
                m_r = off[0]
                m_vec, fb_vec = carry
                bad_r = (m_r < K) | (m_r > CAP)
                fb = lax.cond(bad_r,
                              lambda: _fallback_topk_sum(buf, r, nvreg),
                              lambda: jnp.float32(0.0))
                lane_r = 8 * bufi + r
                m_vec = jnp.where(lanes == lane_r, m_r, m_vec)
                fb_vec = jnp.where(lanes == lane_r, fb, fb_vec)
                return m_vec, fb_vec

            carry = lax.fori_loop(0, 8, abn_row, carry)

            @pl.when(batch + 1 < nbatch)
            def _prefetch(bufi=bufi):
                cp(c0 + 16 + 8 * bufi, bufi).start()

        m_vec, fb_vec = carry

        groups = (jnp.minimum(jnp.max(m_vec), CAP) + 7) >> 3

        def cand_pass(fn, init):
            def body(j, accs):
                return tuple(
                    fn(cand[j * 8 + u], acc) for u, acc in enumerate(accs)
                )

            accs = plsc.parallel_loop(0, groups, unroll=1,
                                      carry=(init,) * 8)(body)
            out = accs[0]
            for a in accs[1:]:
                out = jax.tree.map(lambda x, y: x + y, out, a)
            return out

        def bit_iter(_, carry):
            lo, hi = carry
            mid = lo + ((hi - lo + 1) >> 1)

            def cnt_fn(row, cnt):
                b = lax.bitcast_convert_type(row, jnp.int32)
                return cnt + jnp.where(b >= mid, 1, 0)

            cnt = cand_pass(cnt_fn, jnp.zeros((16,), jnp.int32))
            ge = cnt >= K
            return jnp.where(ge, mid, lo), jnp.where(ge, hi, mid - 1)

        lo, hi = lax.fori_loop(0, 19, bit_iter,
                               (jnp.full((16,), T0_BITS, jnp.int32),
                                jnp.full((16,), ONE_BITS - 1, jnp.int32)))
        kth = lax.bitcast_convert_type(lo, jnp.float32)

        def corr_fn(row, carry):
            s, c = carry
            b = lax.bitcast_convert_type(row, jnp.int32)
            g = b > lo
            return s + jnp.where(g, row, 0.0), c + g.astype(jnp.int32)

        s, c = cand_pass(corr_fn, (jnp.zeros((16,), jnp.float32),
                                   jnp.zeros((16,), jnp.int32)))
        deg_b = s + (K - c).astype(jnp.float32) * kth

        bad = (m_vec < K) | (m_vec > CAP)
        degbuf[pl.ds(rpw + batch * 16, 16)] = jnp.where(bad, fb_vec, deg_b)
        return 0

    lax.fori_loop(0, nbatch, abn_batch, 0)

    pltpu.sync_copy(degbuf.at[pl.ds(0, rpw)],
                    deg_hbm.at[pl.ds(nor0, rpw)])
    pltpu.sync_copy(degbuf.at[pl.ds(rpw, rpw)],
                    deg_hbm.at[pl.ds(abn0, rpw)])


def _loss_kernel(deg_ref, out_ref, *, b2):
    deg = deg_ref[...]
    b, c = deg.shape
    mean = jnp.mean(deg, axis=1, keepdims=True)
    d = deg - mean
    var = jnp.sum(d * d, axis=1, keepdims=True) / (c - 1)
    sign = jnp.where(lax.broadcasted_iota(jnp.int32, (b, 1), 0) < b2, 1.0, -1.0)
    out_ref[...] = jnp.sum(var * sign, axis=(0, 1), keepdims=True) / b2


def kernel(features):
    b, c, t = features.shape
    b2 = b // 2
    nrows = b * c
    feat2d = jnp.reshape(features, (nrows, t))

    rpw = (nrows // 2) // NW
    assert rpw % 16 == 0 and t % 128 == 0
    mesh = plsc.VectorSubcoreMesh(core_axis_name="c", subcore_axis_name="s",
                                  num_cores=2, num_subcores=16)
    sc_deg = pl.kernel(
        functools.partial(_sc_deg_body, nvreg=t // 16, rpw=rpw),
        out_type=jax.ShapeDtypeStruct((nrows,), jnp.float32),
        mesh=mesh,
        scratch_types=[
            pltpu.VMEM((2, 8, t), jnp.float32),
            pltpu.VMEM((CAP, 16), jnp.float32),
            pltpu.VMEM((2 * rpw,), jnp.float32),
            pltpu.SemaphoreType.DMA,
            pltpu.SemaphoreType.DMA,
        ],
        compiler_params=pltpu.CompilerParams(needs_layout_passes=False),
    )
    deg = jnp.reshape(sc_deg(feat2d), (b, c))

    loss = pl.pallas_call(
        functools.partial(_loss_kernel, b2=b2),
        out_shape=jax.ShapeDtypeStruct((1, 1), jnp.float32),
    )(deg)
    return jnp.reshape(loss, ())


# speedup vs baseline: 3.8757x; 1.0022x over previous
import functools

import jax
import jax.numpy as jnp
from jax import lax
from jax.experimental import pallas as pl
from jax.experimental.pallas import tpu as pltpu
from jax.experimental.pallas import tpu_sc as plsc

K = 64
THRESHOLD = 0.5
T0_BITS = 0x3F780000
HALF_BITS = 0x3F000000
ONE_BITS = 0x3F800000
CAP = 256

NW = 32


def _row_pass(buf, r, fn, init, nvreg):
    def body(j, carry):
        for u in range(8):
            v = buf[r, pl.ds((j * 8 + u) * 16, 16)]
            carry = fn(v, carry)
        return carry

    return lax.fori_loop(0, nvreg // 8, body, init)


def _row_pass8(buf, r, fn, init1, nvreg):
    def body(j, accs):
        return tuple(
            fn(buf[r, pl.ds((j * 8 + u) * 16, 16)], acc)
            for u, acc in enumerate(accs)
        )

    accs = plsc.parallel_loop(0, nvreg // 8, unroll=2,
                              carry=(init1,) * 8)(body)
    out = accs[0]
    for a in accs[1:]:
        out = jax.tree.map(lambda x, y: x + y, out, a)
    return out


def _masked_sum_scalar(buf, r, nvreg):
    def fn(v, acc):
        return acc + jnp.where(v >= THRESHOLD, v, 0.0)

    acc = _row_pass8(buf, r, fn, jnp.zeros((16,), jnp.float32), nvreg)
    return jnp.sum(acc)


def _fallback_topk_sum(buf, r, nvreg):
    def fn(v, carry):
        s, c = carry
        m = v >= THRESHOLD
        return s + jnp.where(m, v, 0.0), c + m.astype(jnp.int32)

    s5, c5 = _row_pass(buf, r, fn, (jnp.zeros((16,), jnp.float32),
                                    jnp.zeros((16,), jnp.int32)), nvreg)
    count5 = jnp.sum(c5)
    sum5 = jnp.sum(s5)

    def search(_):
        lo, hi = HALF_BITS, ONE_BITS - 1

        def bit_iter(_, carry):
            lo, hi = carry
            mid = lo + ((hi - lo + 1) >> 1)

            def fn(v, cnt):
                b = lax.bitcast_convert_type(v, jnp.int32)
                return cnt + jnp.where(b >= mid, 1, 0)

            cnt = jnp.sum(_row_pass(buf, r, fn, jnp.zeros((16,), jnp.int32),
                                    nvreg))
            ge = cnt >= K
            return jnp.where(ge, mid, lo), jnp.where(ge, hi, mid - 1)

        lo, hi = lax.fori_loop(0, 23, bit_iter, (lo, hi))
        kth = lax.bitcast_convert_type(lo, jnp.float32)

        def fn2(v, carry):
            s, c = carry
            b = lax.bitcast_convert_type(v, jnp.int32)
            g = b > lo
            return s + jnp.where(g, v, 0.0), c + g.astype(jnp.int32)

        s, c = _row_pass(buf, r, fn2, (jnp.zeros((16,), jnp.float32),
                                       jnp.zeros((16,), jnp.int32)), nvreg)
        return jnp.sum(s) + (K - jnp.sum(c)).astype(jnp.float32) * kth

    return jnp.where(count5 < K, sum5, search(None))


def _sc_deg_body(feat_hbm, deg_hbm, rowbuf, cand, degbuf, sem0, sem1,
                 *, nvreg, rpw):
    wid = lax.axis_index("s") * 2 + lax.axis_index("c")
    nor0 = wid * rpw
    abn0 = NW * rpw + wid * rpw
    lanes = lax.iota(jnp.int32, 16)
    t0f = lax.bitcast_convert_type(jnp.full((16,), T0_BITS, jnp.int32),
                                   jnp.float32)

    nbatch = rpw // 16
    sems = (sem0, sem1)

    def cp(row0, bufi):
        return pltpu.make_async_copy(feat_hbm.at[pl.ds(row0, 8)],
                                     rowbuf.at[bufi], sems[bufi])

    cp(nor0, 0).start()
    cp(nor0 + 8, 1).start()

    def nor_batch(batch, _):
        c0 = nor0 + batch * 16
        degs = jnp.zeros((16,), jnp.float32)
        for bufi in range(2):
            cp(c0 + 8 * bufi, bufi).wait()

            def nor_row(r, degs, bufi=bufi):
                s = _masked_sum_scalar(rowbuf.at[bufi], r, nvreg)
                return jnp.where(lanes == 8 * bufi + r, s, degs)

            degs = lax.fori_loop(0, 8, nor_row, degs)

            @pl.when(batch + 1 < nbatch)
            def _prefetch(bufi=bufi):
                cp(c0 + 16 + 8 * bufi, bufi).start()

        degbuf[pl.ds(batch * 16, 16)] = degs
        return 0

    with jax.named_scope("nor_half"):
        lax.fori_loop(0, nbatch, nor_batch, 0)

    cp(abn0, 0).start()
    cp(abn0 + 8, 1).start()

    def abn_batch(batch, _):
        c0 = abn0 + batch * 16

        @plsc.parallel_loop(0, CAP, unroll=8)
        def _zero(j):
            cand[j] = jnp.zeros((16,), jnp.float32)

        carry = (jnp.zeros((16,), jnp.int32), jnp.zeros((16,), jnp.float32))
        for bufi in range(2):
            cp(c0 + 8 * bufi, bufi).wait()

            def abn_row(r, carry, bufi=bufi):
                buf = rowbuf.at[bufi]
                col = lanes * 0 + (8 * bufi + r)

                def fn(v, off):
                    m = v >= t0f
                    pc = plsc.all_reduce_population_count(m)
                    pref = plsc.cumsum(m.astype(jnp.int32))
                    pos = off + pref - 1
                    ok = m & (pos < CAP)
                    posc = jnp.clip(pos, 0, CAP - 1)
                    plsc.store_scatter(cand, [posc, col], v, mask=ok)
                    return off + pc

                def fnA(j, off):
                    return fn(buf[r, pl.ds(j * 16, 16)], off)

                off = plsc.parallel_loop(0, nvreg, unroll=8,
                                         carry=jnp.zeros((16,), jnp.int32))(fnA)
                m_r = off[0]
                m_vec, fb_vec = carry
                bad_r = (m_r < K) | (m_r > CAP)
                fb = lax.cond(bad_r,
                              lambda: _fallback_topk_sum(buf, r, nvreg),
                              lambda: jnp.float32(0.0))
                lane_r = 8 * bufi + r
                m_vec = jnp.where(lanes == lane_r, m_r, m_vec)
                fb_vec = jnp.where(lanes == lane_r, fb, fb_vec)
                return m_vec, fb_vec

            carry = lax.fori_loop(0, 8, abn_row, carry)

            @pl.when(batch + 1 < nbatch)
            def _prefetch(bufi=bufi):
                cp(c0 + 16 + 8 * bufi, bufi).start()

        m_vec, fb_vec = carry

        groups = (jnp.minimum(jnp.max(m_vec), CAP) + 7) >> 3

        def cand_pass(fn, init):
            def body(j, accs):
                return tuple(
                    fn(cand[j * 8 + u], acc) for u, acc in enumerate(accs)
                )

            accs = plsc.parallel_loop(0, groups, unroll=1,
                                      carry=(init,) * 8)(body)
            out = accs[0]
            for a in accs[1:]:
                out = jax.tree.map(lambda x, y: x + y, out, a)
            return out

        def bit_iter(_, carry):
            lo, hi = carry
            mid = lo + ((hi - lo + 1) >> 1)

            def cnt_fn(row, cnt):
                b = lax.bitcast_convert_type(row, jnp.int32)
                return cnt + jnp.where(b >= mid, 1, 0)

            cnt = cand_pass(cnt_fn, jnp.zeros((16,), jnp.int32))
            ge = cnt >= K
            return jnp.where(ge, mid, lo), jnp.where(ge, hi, mid - 1)

        lo, hi = lax.fori_loop(0, 19, bit_iter,
                               (jnp.full((16,), T0_BITS, jnp.int32),
                                jnp.full((16,), ONE_BITS - 1, jnp.int32)))
        kth = lax.bitcast_convert_type(lo, jnp.float32)

        def corr_fn(row, carry):
            s, c = carry
            b = lax.bitcast_convert_type(row, jnp.int32)
            g = b > lo
            return s + jnp.where(g, row, 0.0), c + g.astype(jnp.int32)

        s, c = cand_pass(corr_fn, (jnp.zeros((16,), jnp.float32),
                                   jnp.zeros((16,), jnp.int32)))
        deg_b = s + (K - c).astype(jnp.float32) * kth

        bad = (m_vec < K) | (m_vec > CAP)
        degbuf[pl.ds(rpw + batch * 16, 16)] = jnp.where(bad, fb_vec, deg_b)
        return 0

    with jax.named_scope("abn_half"):
        lax.fori_loop(0, nbatch, abn_batch, 0)

    pltpu.sync_copy(degbuf.at[pl.ds(0, rpw)],
                    deg_hbm.at[pl.ds(nor0, rpw)])
    pltpu.sync_copy(degbuf.at[pl.ds(rpw, rpw)],
                    deg_hbm.at[pl.ds(abn0, rpw)])


def _loss_kernel(deg_ref, out_ref, *, b2):
    deg = deg_ref[...]
    b, c = deg.shape
    mean = jnp.mean(deg, axis=1, keepdims=True)
    d = deg - mean
    var = jnp.sum(d * d, axis=1, keepdims=True) / (c - 1)
    sign = jnp.where(lax.broadcasted_iota(jnp.int32, (b, 1), 0) < b2, 1.0, -1.0)
    out_ref[...] = jnp.sum(var * sign, axis=(0, 1), keepdims=True) / b2


def kernel(features):
    b, c, t = features.shape
    b2 = b // 2
    nrows = b * c
    feat2d = jnp.reshape(features, (nrows, t))

    rpw = (nrows // 2) // NW
    assert rpw % 16 == 0 and t % 128 == 0
    mesh = plsc.VectorSubcoreMesh(core_axis_name="c", subcore_axis_name="s",
                                  num_cores=2, num_subcores=16)
    sc_deg = pl.kernel(
        functools.partial(_sc_deg_body, nvreg=t // 16, rpw=rpw),
        out_type=jax.ShapeDtypeStruct((nrows,), jnp.float32),
        mesh=mesh,
        scratch_types=[
            pltpu.VMEM((2, 8, t), jnp.float32),
            pltpu.VMEM((CAP, 16), jnp.float32),
            pltpu.VMEM((2 * rpw,), jnp.float32),
            pltpu.SemaphoreType.DMA,
            pltpu.SemaphoreType.DMA,
        ],
        compiler_params=pltpu.CompilerParams(needs_layout_passes=False),
    )
    deg = jnp.reshape(sc_deg(feat2d), (b, c))

    loss = pl.pallas_call(
        functools.partial(_loss_kernel, b2=b2),
        out_shape=jax.ShapeDtypeStruct((1, 1), jnp.float32),
    )(deg)
    return jnp.reshape(loss, ())


# speedup vs baseline: 4.6452x; 1.1985x over previous
import functools

import jax
import jax.numpy as jnp
from jax import lax
from jax.experimental import pallas as pl
from jax.experimental.pallas import tpu as pltpu
from jax.experimental.pallas import tpu_sc as plsc

K = 64
THRESHOLD = 0.5
T0_BITS = 0x3F780000
HALF_BITS = 0x3F000000
ONE_BITS = 0x3F800000
CAP = 256

NW = 32


def _row_pass(buf, r, fn, init, nvreg):
    def body(j, carry):
        for u in range(8):
            v = buf[r, pl.ds((j * 8 + u) * 16, 16)]
            carry = fn(v, carry)
        return carry

    return lax.fori_loop(0, nvreg // 8, body, init)


def _row_pass8(buf, r, fn, init1, nvreg):
    def body(j, accs):
        return tuple(
            fn(buf[r, pl.ds((j * 8 + u) * 16, 16)], acc)
            for u, acc in enumerate(accs)
        )

    accs = plsc.parallel_loop(0, nvreg // 8, unroll=2,
                              carry=(init1,) * 8)(body)
    out = accs[0]
    for a in accs[1:]:
        out = jax.tree.map(lambda x, y: x + y, out, a)
    return out


def _masked_sum_scalar(buf, r, nvreg):
    def fn(v, acc):
        return acc + jnp.where(v >= THRESHOLD, v, 0.0)

    acc = _row_pass8(buf, r, fn, jnp.zeros((16,), jnp.float32), nvreg)
    return jnp.sum(acc)


def _fallback_topk_sum(buf, r, nvreg):
    def fn(v, carry):
        s, c = carry
        m = v >= THRESHOLD
        return s + jnp.where(m, v, 0.0), c + m.astype(jnp.int32)

    s5, c5 = _row_pass(buf, r, fn, (jnp.zeros((16,), jnp.float32),
                                    jnp.zeros((16,), jnp.int32)), nvreg)
    count5 = jnp.sum(c5)
    sum5 = jnp.sum(s5)

    def search(_):
        lo, hi = HALF_BITS, ONE_BITS - 1

        def bit_iter(_, carry):
            lo, hi = carry
            mid = lo + ((hi - lo + 1) >> 1)

            def fn(v, cnt):
                b = lax.bitcast_convert_type(v, jnp.int32)
                return cnt + jnp.where(b >= mid, 1, 0)

            cnt = jnp.sum(_row_pass(buf, r, fn, jnp.zeros((16,), jnp.int32),
                                    nvreg))
            ge = cnt >= K
            return jnp.where(ge, mid, lo), jnp.where(ge, hi, mid - 1)

        lo, hi = lax.fori_loop(0, 23, bit_iter, (lo, hi))
        kth = lax.bitcast_convert_type(lo, jnp.float32)

        def fn2(v, carry):
            s, c = carry
            b = lax.bitcast_convert_type(v, jnp.int32)
            g = b > lo
            return s + jnp.where(g, v, 0.0), c + g.astype(jnp.int32)

        s, c = _row_pass(buf, r, fn2, (jnp.zeros((16,), jnp.float32),
                                       jnp.zeros((16,), jnp.int32)), nvreg)
        return jnp.sum(s) + (K - jnp.sum(c)).astype(jnp.float32) * kth

    return jnp.where(count5 < K, sum5, search(None))


def _sc_deg_body(feat_hbm, deg_hbm, rowbuf, cand, degbuf, sem0, sem1,
                 *, nvreg, rpw, abn_base):
    wid = lax.axis_index("s") * 2 + lax.axis_index("c")
    abn0 = abn_base + wid * rpw
    lanes = lax.iota(jnp.int32, 16)
    t0f = lax.bitcast_convert_type(jnp.full((16,), T0_BITS, jnp.int32),
                                   jnp.float32)

    nbatch = rpw // 16
    sems = (sem0, sem1)

    def cp(row0, bufi):
        return pltpu.make_async_copy(feat_hbm.at[pl.ds(row0, 8)],
                                     rowbuf.at[bufi], sems[bufi])

    cp(abn0, 0).start()
    cp(abn0 + 8, 1).start()

    def abn_batch(batch, _):
        c0 = abn0 + batch * 16

        @plsc.parallel_loop(0, CAP, unroll=8)
        def _zero(j):
            cand[j] = jnp.zeros((16,), jnp.float32)

        carry = (jnp.zeros((16,), jnp.int32), jnp.zeros((16,), jnp.float32))
        for bufi in range(2):
            cp(c0 + 8 * bufi, bufi).wait()

            def abn_row(r, carry, bufi=bufi):
                buf = rowbuf.at[bufi]
                col = lanes * 0 + (8 * bufi + r)

                def fn(v, off):
                    m = v >= t0f
                    pc = plsc.all_reduce_population_count(m)
                    pref = plsc.cumsum(m.astype(jnp.int32))
                    pos = off + pref - 1
                    ok = m & (pos < CAP)
                    posc = jnp.clip(pos, 0, CAP - 1)
                    plsc.store_scatter(cand, [posc, col], v, mask=ok)
                    return off + pc

                def fnA(j, off):
                    return fn(buf[r, pl.ds(j * 16, 16)], off)

                off = plsc.parallel_loop(0, nvreg, unroll=8,
                                         carry=jnp.zeros((16,), jnp.int32))(fnA)
                m_r = off[0]
                m_vec, fb_vec = carry
                bad_r = (m_r < K) | (m_r > CAP)
                fb = lax.cond(bad_r,
                              lambda: _fallback_topk_sum(buf, r, nvreg),
                              lambda: jnp.float32(0.0))
                lane_r = 8 * bufi + r
                m_vec = jnp.where(lanes == lane_r, m_r, m_vec)
                fb_vec = jnp.where(lanes == lane_r, fb, fb_vec)
                return m_vec, fb_vec

            carry = lax.fori_loop(0, 8, abn_row, carry)

            @pl.when(batch + 1 < nbatch)
            def _prefetch(bufi=bufi):
                cp(c0 + 16 + 8 * bufi, bufi).start()

        m_vec, fb_vec = carry

        groups = (jnp.minimum(jnp.max(m_vec), CAP) + 7) >> 3

        def cand_pass(fn, init):
            def body(j, accs):
                return tuple(
                    fn(cand[j * 8 + u], acc) for u, acc in enumerate(accs)
                )

            accs = plsc.parallel_loop(0, groups, unroll=1,
                                      carry=(init,) * 8)(body)
            out = accs[0]
            for a in accs[1:]:
                out = jax.tree.map(lambda x, y: x + y, out, a)
            return out

        def bit_iter(_, carry):
            lo, hi = carry
            mid = lo + ((hi - lo + 1) >> 1)

            def cnt_fn(row, cnt):
                b = lax.bitcast_convert_type(row, jnp.int32)
                return cnt + jnp.where(b >= mid, 1, 0)

            cnt = cand_pass(cnt_fn, jnp.zeros((16,), jnp.int32))
            ge = cnt >= K
            return jnp.where(ge, mid, lo), jnp.where(ge, hi, mid - 1)

        lo, hi = lax.fori_loop(0, 19, bit_iter,
                               (jnp.full((16,), T0_BITS, jnp.int32),
                                jnp.full((16,), ONE_BITS - 1, jnp.int32)))
        kth = lax.bitcast_convert_type(lo, jnp.float32)

        def corr_fn(row, carry):
            s, c = carry
            b = lax.bitcast_convert_type(row, jnp.int32)
            g = b > lo
            return s + jnp.where(g, row, 0.0), c + g.astype(jnp.int32)

        s, c = cand_pass(corr_fn, (jnp.zeros((16,), jnp.float32),
                                   jnp.zeros((16,), jnp.int32)))
        deg_b = s + (K - c).astype(jnp.float32) * kth

        bad = (m_vec < K) | (m_vec > CAP)
        degbuf[pl.ds(batch * 16, 16)] = jnp.where(bad, fb_vec, deg_b)
        return 0

    lax.fori_loop(0, nbatch, abn_batch, 0)

    pltpu.sync_copy(degbuf, deg_hbm.at[pl.ds(wid * rpw, rpw)])


def _nor_deg_kernel(x_ref, deg_ref):
    x = x_ref[0]
    masked = jnp.where(x >= THRESHOLD, x, 0.0)
    deg_ref[0, 0, :] = jnp.sum(masked, axis=1)


def _loss_kernel(degn_ref, dega_ref, out_ref, *, b2):
    def var_rows(deg):
        mean = jnp.mean(deg, axis=1, keepdims=True)
        d = deg - mean
        return jnp.sum(d * d, axis=1, keepdims=True) / (deg.shape[1] - 1)

    vn = var_rows(degn_ref[...])
    va = var_rows(dega_ref[...])
    out_ref[...] = (jnp.sum(vn, axis=(0, 1), keepdims=True)
                    - jnp.sum(va, axis=(0, 1), keepdims=True)) / b2


def kernel(features):
    b, c, t = features.shape
    b2 = b // 2
    nabn = b2 * c
    feat2d = jnp.reshape(features, (b * c, t))

    rpw = nabn // NW
    assert rpw % 16 == 0 and t % 128 == 0
    mesh = plsc.VectorSubcoreMesh(core_axis_name="c", subcore_axis_name="s",
                                  num_cores=2, num_subcores=16)
    sc_deg = pl.kernel(
        functools.partial(_sc_deg_body, nvreg=t // 16, rpw=rpw, abn_base=nabn),
        out_type=jax.ShapeDtypeStruct((nabn,), jnp.float32),
        mesh=mesh,
        scratch_types=[
            pltpu.VMEM((2, 8, t), jnp.float32),
            pltpu.VMEM((CAP, 16), jnp.float32),
            pltpu.VMEM((rpw,), jnp.float32),
            pltpu.SemaphoreType.DMA,
            pltpu.SemaphoreType.DMA,
        ],
        compiler_params=pltpu.CompilerParams(needs_layout_passes=False),
    )
    deg_abn = sc_deg(feat2d)

    deg_nor = pl.pallas_call(
        _nor_deg_kernel,
        grid=(b2,),
        in_specs=[pl.BlockSpec((1, c, t), lambda i: (i, 0, 0))],
        out_specs=pl.BlockSpec((1, 1, c), lambda i: (i, 0, 0)),
        out_shape=jax.ShapeDtypeStruct((b2, 1, c), jnp.float32),
    )(features)

    loss = pl.pallas_call(
        functools.partial(_loss_kernel, b2=b2),
        out_shape=jax.ShapeDtypeStruct((1, 1), jnp.float32),
    )(jnp.reshape(deg_nor, (b2, c)), jnp.reshape(deg_abn, (b2, c)))
    return jnp.reshape(loss, ())


# speedup vs baseline: 4.7232x; 1.0168x over previous
import functools

import jax
import jax.numpy as jnp
from jax import lax
from jax.experimental import pallas as pl
from jax.experimental.pallas import tpu as pltpu
from jax.experimental.pallas import tpu_sc as plsc

K = 64
THRESHOLD = 0.5
T0_BITS = 0x3F780000
HALF_BITS = 0x3F000000
ONE_BITS = 0x3F800000
CAP = 256

NW = 32


def _row_pass(buf, r, fn, init, nvreg):
    def body(j, carry):
        for u in range(8):
            v = buf[r, pl.ds((j * 8 + u) * 16, 16)]
            carry = fn(v, carry)
        return carry

    return lax.fori_loop(0, nvreg // 8, body, init)


def _row_pass8(buf, r, fn, init1, nvreg):
    def body(j, accs):
        return tuple(
            fn(buf[r, pl.ds((j * 8 + u) * 16, 16)], acc)
            for u, acc in enumerate(accs)
        )

    accs = plsc.parallel_loop(0, nvreg // 8, unroll=2,
                              carry=(init1,) * 8)(body)
    out = accs[0]
    for a in accs[1:]:
        out = jax.tree.map(lambda x, y: x + y, out, a)
    return out


def _masked_sum_scalar(buf, r, nvreg):
    def fn(v, acc):
        return acc + jnp.where(v >= THRESHOLD, v, 0.0)

    acc = _row_pass8(buf, r, fn, jnp.zeros((16,), jnp.float32), nvreg)
    return jnp.sum(acc)


def _fallback_topk_sum(buf, r, nvreg):
    def fn(v, carry):
        s, c = carry
        m = v >= THRESHOLD
        return s + jnp.where(m, v, 0.0), c + m.astype(jnp.int32)

    s5, c5 = _row_pass(buf, r, fn, (jnp.zeros((16,), jnp.float32),
                                    jnp.zeros((16,), jnp.int32)), nvreg)
    count5 = jnp.sum(c5)
    sum5 = jnp.sum(s5)

    def search(_):
        lo, hi = HALF_BITS, ONE_BITS - 1

        def bit_iter(_, carry):
            lo, hi = carry
            mid = lo + ((hi - lo + 1) >> 1)

            def fn(v, cnt):
                b = lax.bitcast_convert_type(v, jnp.int32)
                return cnt + jnp.where(b >= mid, 1, 0)

            cnt = jnp.sum(_row_pass(buf, r, fn, jnp.zeros((16,), jnp.int32),
                                    nvreg))
            ge = cnt >= K
            return jnp.where(ge, mid, lo), jnp.where(ge, hi, mid - 1)

        lo, hi = lax.fori_loop(0, 23, bit_iter, (lo, hi))
        kth = lax.bitcast_convert_type(lo, jnp.float32)

        def fn2(v, carry):
            s, c = carry
            b = lax.bitcast_convert_type(v, jnp.int32)
            g = b > lo
            return s + jnp.where(g, v, 0.0), c + g.astype(jnp.int32)

        s, c = _row_pass(buf, r, fn2, (jnp.zeros((16,), jnp.float32),
                                       jnp.zeros((16,), jnp.int32)), nvreg)
        return jnp.sum(s) + (K - jnp.sum(c)).astype(jnp.float32) * kth

    return jnp.where(count5 < K, sum5, search(None))


def _sc_deg_body(feat_hbm, deg_hbm, rowbuf, cand, degbuf, sem0, sem1,
                 *, nvreg, rpw, abn_base):
    wid = lax.axis_index("s") * 2 + lax.axis_index("c")
    abn0 = abn_base + wid * rpw
    lanes = lax.iota(jnp.int32, 16)
    t0f = lax.bitcast_convert_type(jnp.full((16,), T0_BITS, jnp.int32),
                                   jnp.float32)

    nbatch = rpw // 16
    sems = (sem0, sem1)

    def cp(row0, bufi):
        return pltpu.make_async_copy(feat_hbm.at[pl.ds(row0, 8)],
                                     rowbuf.at[bufi], sems[bufi])

    cp(abn0, 0).start()
    cp(abn0 + 8, 1).start()

    def abn_batch(batch, _):
        c0 = abn0 + batch * 16

        @plsc.parallel_loop(0, CAP, unroll=8)
        def _zero(j):
            cand[j] = jnp.zeros((16,), jnp.float32)

        carry = (jnp.zeros((16,), jnp.int32), jnp.zeros((16,), jnp.float32))
        for bufi in range(2):
            cp(c0 + 8 * bufi, bufi).wait()

            def abn_row(r, carry, bufi=bufi):
                buf = rowbuf.at[bufi]
                col = lanes * 0 + (8 * bufi + r)

                def fn(v, off):
                    m = v >= t0f
                    pc = plsc.all_reduce_population_count(m)
                    pref = plsc.cumsum(m.astype(jnp.int32))
                    pos = off + pref - 1
                    ok = m & (pos < CAP)
                    posc = jnp.clip(pos, 0, CAP - 1)
                    plsc.store_scatter(cand, [posc, col], v, mask=ok)
                    return off + pc

                def fnA(j, off):
                    return fn(buf[r, pl.ds(j * 16, 16)], off)

                off = plsc.parallel_loop(0, nvreg, unroll=8,
                                         carry=jnp.zeros((16,), jnp.int32))(fnA)
                m_r = off[0]
                m_vec, fb_vec = carry
                bad_r = (m_r < K) | (m_r > CAP)
                fb = lax.cond(bad_r,
                              lambda: _fallback_topk_sum(buf, r, nvreg),
                              lambda: jnp.float32(0.0))
                lane_r = 8 * bufi + r
                m_vec = jnp.where(lanes == lane_r, m_r, m_vec)
                fb_vec = jnp.where(lanes == lane_r, fb, fb_vec)
                return m_vec, fb_vec

            carry = lax.fori_loop(0, 8, abn_row, carry)

            @pl.when(batch + 1 < nbatch)
            def _prefetch(bufi=bufi):
                cp(c0 + 16 + 8 * bufi, bufi).start()

        m_vec, fb_vec = carry

        groups = (jnp.minimum(jnp.max(m_vec), CAP) + 7) >> 3

        def cand_pass(fn, init):
            def body(j, accs):
                return tuple(
                    fn(cand[j * 8 + u], acc) for u, acc in enumerate(accs)
                )

            accs = plsc.parallel_loop(0, groups, unroll=2,
                                      carry=(init,) * 8)(body)
            out = accs[0]
            for a in accs[1:]:
                out = jax.tree.map(lambda x, y: x + y, out, a)
            return out

        def bit_iter(_, carry):
            lo, hi = carry
            mid = lo + ((hi - lo + 1) >> 1)

            def cnt_fn(row, cnt):
                b = lax.bitcast_convert_type(row, jnp.int32)
                return cnt + jnp.where(b >= mid, 1, 0)

            cnt = cand_pass(cnt_fn, jnp.zeros((16,), jnp.int32))
            ge = cnt >= K
            return jnp.where(ge, mid, lo), jnp.where(ge, hi, mid - 1)

        lo, hi = lax.fori_loop(0, 19, bit_iter,
                               (jnp.full((16,), T0_BITS, jnp.int32),
                                jnp.full((16,), ONE_BITS - 1, jnp.int32)))
        kth = lax.bitcast_convert_type(lo, jnp.float32)

        def corr_fn(row, carry):
            s, c = carry
            b = lax.bitcast_convert_type(row, jnp.int32)
            g = b > lo
            return s + jnp.where(g, row, 0.0), c + g.astype(jnp.int32)

        s, c = cand_pass(corr_fn, (jnp.zeros((16,), jnp.float32),
                                   jnp.zeros((16,), jnp.int32)))
        deg_b = s + (K - c).astype(jnp.float32) * kth

        bad = (m_vec < K) | (m_vec > CAP)
        degbuf[pl.ds(batch * 16, 16)] = jnp.where(bad, fb_vec, deg_b)
        return 0

    lax.fori_loop(0, nbatch, abn_batch, 0)

    pltpu.sync_copy(degbuf, deg_hbm.at[pl.ds(wid * rpw, rpw)])


def _nor_deg_kernel(x_ref, deg_ref):
    x = x_ref[0]
    masked = jnp.where(x >= THRESHOLD, x, 0.0)
    deg_ref[0, 0, :] = jnp.sum(masked, axis=1)


def _loss_kernel(degn_ref, dega_ref, out_ref, *, b2):
    def var_rows(deg):
        mean = jnp.mean(deg, axis=1, keepdims=True)
        d = deg - mean
        return jnp.sum(d * d, axis=1, keepdims=True) / (deg.shape[1] - 1)

    vn = var_rows(degn_ref[...])
    va = var_rows(dega_ref[...])
    out_ref[...] = (jnp.sum(vn, axis=(0, 1), keepdims=True)
                    - jnp.sum(va, axis=(0, 1), keepdims=True)) / b2


def kernel(features):
    b, c, t = features.shape
    b2 = b // 2
    nabn = b2 * c
    feat2d = jnp.reshape(features, (b * c, t))

    rpw = nabn // NW
    assert rpw % 16 == 0 and t % 128 == 0
    mesh = plsc.VectorSubcoreMesh(core_axis_name="c", subcore_axis_name="s",
                                  num_cores=2, num_subcores=16)
    sc_deg = pl.kernel(
        functools.partial(_sc_deg_body, nvreg=t // 16, rpw=rpw, abn_base=nabn),
        out_type=jax.ShapeDtypeStruct((nabn,), jnp.float32),
        mesh=mesh,
        scratch_types=[
            pltpu.VMEM((2, 8, t), jnp.float32),
            pltpu.VMEM((CAP, 16), jnp.float32),
            pltpu.VMEM((rpw,), jnp.float32),
            pltpu.SemaphoreType.DMA,
            pltpu.SemaphoreType.DMA,
        ],
        compiler_params=pltpu.CompilerParams(needs_layout_passes=False),
    )
    deg_abn = sc_deg(feat2d)

    deg_nor = pl.pallas_call(
        _nor_deg_kernel,
        grid=(b2,),
        in_specs=[pl.BlockSpec((1, c, t), lambda i: (i, 0, 0))],
        out_specs=pl.BlockSpec((1, 1, c), lambda i: (i, 0, 0)),
        out_shape=jax.ShapeDtypeStruct((b2, 1, c), jnp.float32),
    )(features)

    loss = pl.pallas_call(
        functools.partial(_loss_kernel, b2=b2),
        out_shape=jax.ShapeDtypeStruct((1, 1), jnp.float32),
    )(jnp.reshape(deg_nor, (b2, c)), jnp.reshape(deg_abn, (b2, c)))
    return jnp.reshape(loss, ())
